# baseline (speedup 1.0000x reference)
import functools

import jax
import jax.numpy as jnp
from jax import lax
from jax.experimental import pallas as pl
from jax.experimental.pallas import tpu as pltpu
from jax.experimental.pallas import tpu_sc as plsc

NN = 10000
EE = 320000
DD = 128
NRBF_K = 8
RMAX_V = 10.0
C0 = (2.0 / RMAX_V) ** 0.5

BE = 2000
BN = 2000


def _silu(x):
    return x * jax.nn.sigmoid(x)


def _dsilu(x):
    s = jax.nn.sigmoid(x)
    return s * (1.0 + x * (1.0 - s))


def _geom(v):
    d2 = jnp.sum(v * v, axis=1, keepdims=True)
    r = jnp.sqrt(d2 + 1e-12)
    x = r / RMAX_V
    x2 = x * x
    x4 = x2 * x2
    x5 = x4 * x
    x6 = x4 * x2
    x7 = x6 * x
    x8 = x4 * x4
    lt1 = x < 1.0
    env = jnp.where(lt1, 1.0 - 28.0 * x6 + 48.0 * x7 - 21.0 * x8, 0.0)
    denv = jnp.where(lt1, -168.0 * x5 + 336.0 * x6 - 168.0 * x7, 0.0)
    kk = (lax.broadcasted_iota(jnp.int32, (1, NRBF_K), 1) + 1
          ).astype(jnp.float32)
    arg = kk * (jnp.pi * x)
    sinv = jnp.sin(arg)
    rinv = 1.0 / (r + 1e-12)
    rbf = (C0 * env * rinv) * sinv
    return r, x, env, denv, kk, arg, sinv, rinv, rbf


def _tc_radial_body(vec_ref, w1_ref, w2_ref, out_ref):
    v = vec_ref[...]
    _, _, _, _, _, _, _, _, rbf = _geom(v)
    z1 = jnp.dot(rbf, w1_ref[...], preferred_element_type=jnp.float32)
    a1 = _silu(z1)
    out_ref[...] = jnp.dot(a1, w2_ref[...], preferred_element_type=jnp.float32)


def _tc_rback_body(vec_ref, grad_ref, w1_ref, w2_ref, out_ref):
    v = vec_ref[...]
    r, x, env, denv, kk, arg, sinv, rinv, rbf = _geom(v)
    w1 = w1_ref[...]
    w2 = w2_ref[...]
    z1 = jnp.dot(rbf, w1, preferred_element_type=jnp.float32)
    g_a1 = lax.dot_general(grad_ref[...], w2, (((1,), (1,)), ((), ())),
                           preferred_element_type=jnp.float32)
    g_z1 = g_a1 * _dsilu(z1)
    g_rbf = lax.dot_general(g_z1, w1, (((1,), (1,)), ((), ())),
                            preferred_element_type=jnp.float32)
    cosv = jnp.cos(arg)
    drbf_dr = (C0 * env) * ((kk * (jnp.pi / RMAX_V)) * cosv * rinv
                            - sinv * (rinv * rinv)) \
        + (C0 * sinv * rinv) * (denv / RMAX_V)
    g_r = jnp.sum(g_rbf * drbf_dr, axis=1, keepdims=True)
    out_ref[...] = (g_r / r) * v


def _tc_embed_body(sp_ref, we_ref, h_ref):
    sp = sp_ref[...]
    ids = lax.broadcasted_iota(jnp.int32, (1, 16), 1)
    onehot = (sp == ids).astype(jnp.float32)
    h_ref[...] = jnp.dot(onehot, we_ref[...], preferred_element_type=jnp.float32)


def _tc_node1_body(m1a_ref, m1b_ref, h_ref, u1_ref, m1_ref, h1_ref):
    m1 = m1a_ref[...] + m1b_ref[...]
    h1 = h_ref[...] + jnp.dot(_silu(m1), u1_ref[...],
                              preferred_element_type=jnp.float32)
    m1_ref[...] = m1
    h1_ref[...] = h1


def _tc_node2_body(m2a_ref, m2b_ref, h1_ref, sp_ref, u2_ref, wr_ref, ae_ref,
                   gm2_ref, en_ref):
    m2 = m2a_ref[...] + m2b_ref[...]
    u2 = u2_ref[...]
    wr = wr_ref[...]
    h2 = h1_ref[...] + jnp.dot(_silu(m2), u2, preferred_element_type=jnp.float32)
    sp = sp_ref[...]
    ids = lax.broadcasted_iota(jnp.int32, (1, 16), 1)
    onehot = (sp == ids).astype(jnp.float32)
    e_block = jnp.sum(h2 * wr) + jnp.sum(
        jnp.dot(onehot, ae_ref[...], preferred_element_type=jnp.float32))
    v2 = lax.dot_general(wr, u2, (((1,), (1,)), ((), ())),
                         preferred_element_type=jnp.float32)
    gm2_ref[...] = v2 * _dsilu(m2)

    @pl.when(pl.program_id(0) == 0)
    def _():
        en_ref[...] = jnp.zeros_like(en_ref[...])
    en_ref[...] = en_ref[...] + e_block


def _tc_gm1_body(gsa_ref, gsb_ref, m1_ref, u1_ref, wr_ref, gm1_ref):
    g_h1 = wr_ref[...] + gsa_ref[...] + gsb_ref[...]
    gm1_ref[...] = lax.dot_general(
        g_h1, u1_ref[...], (((1,), (1,)), ((), ())),
        preferred_element_type=jnp.float32) * _dsilu(m1_ref[...])


def _tc_forces_body(fa_ref, fb_ref, out_ref):
    out_ref[...] = -(fa_ref[...] + fb_ref[...])


def _edge_specs(width):
    return pl.BlockSpec((BE, width), lambda i: (i, 0))


def _node_specs(width):
    return pl.BlockSpec((BN, width), lambda i: (i, 0))


def _full(shape):
    return pl.BlockSpec(shape, lambda i: tuple(0 for _ in shape))


def _tc_radial(vec16, W_r1, W_r2):
    return pl.pallas_call(
        _tc_radial_body,
        grid=(EE // BE,),
        in_specs=[_edge_specs(16), _full((NRBF_K, 64)), _full((64, DD))],
        out_specs=_edge_specs(DD),
        out_shape=jax.ShapeDtypeStruct((EE, DD), jnp.float32),
    )(vec16, W_r1, W_r2)


def _tc_rback(vec16, g_radial, W_r1, W_r2):
    return pl.pallas_call(
        _tc_rback_body,
        grid=(EE // BE,),
        in_specs=[_edge_specs(16), _edge_specs(DD),
                  _full((NRBF_K, 64)), _full((64, DD))],
        out_specs=_edge_specs(16),
        out_shape=jax.ShapeDtypeStruct((EE, 16), jnp.float32),
    )(vec16, g_radial, W_r1, W_r2)


def _tc_embed(sp2d, we_pad):
    return pl.pallas_call(
        _tc_embed_body,
        grid=(NN // BN,),
        in_specs=[_node_specs(1), _full((16, DD))],
        out_specs=_node_specs(DD),
        out_shape=jax.ShapeDtypeStruct((NN, DD), jnp.float32),
    )(sp2d, we_pad)


def _tc_node1(m1a, m1b, h, U1):
    return pl.pallas_call(
        _tc_node1_body,
        grid=(NN // BN,),
        in_specs=[_node_specs(DD), _node_specs(DD), _node_specs(DD),
                  _full((DD, DD))],
        out_specs=[_node_specs(DD)] * 2,
        out_shape=[jax.ShapeDtypeStruct((NN, DD), jnp.float32)] * 2,
    )(m1a, m1b, h, U1)


def _tc_node2(m2a, m2b, h1, sp2d, U2, wr2d, ae16):
    return pl.pallas_call(
        _tc_node2_body,
        grid=(NN // BN,),
        in_specs=[_node_specs(DD), _node_specs(DD), _node_specs(DD),
                  _node_specs(1), _full((DD, DD)), _full((1, DD)),
                  _full((16, 1))],
        out_specs=[_node_specs(DD), _full((1, 1))],
        out_shape=[jax.ShapeDtypeStruct((NN, DD), jnp.float32),
                   jax.ShapeDtypeStruct((1, 1), jnp.float32)],
    )(m2a, m2b, h1, sp2d, U2, wr2d, ae16)


def _tc_gm1(gsa, gsb, m1, U1, wr2d):
    return pl.pallas_call(
        _tc_gm1_body,
        grid=(NN // BN,),
        in_specs=[_node_specs(DD), _node_specs(DD), _node_specs(DD),
                  _full((DD, DD)), _full((1, DD))],
        out_specs=_node_specs(DD),
        out_shape=jax.ShapeDtypeStruct((NN, DD), jnp.float32),
    )(gsa, gsb, m1, U1, wr2d)


def _tc_forces(fa, fb):
    return pl.pallas_call(
        _tc_forces_body,
        grid=(NN // BN,),
        in_specs=[_node_specs(16), _node_specs(16)],
        out_specs=_node_specs(16),
        out_shape=jax.ShapeDtypeStruct((NN, 16), jnp.float32),
    )(fa, fb)


def _sc_geom(pos16, s, t):
    return pos16[t] - pos16[s]


def _sc_msg(table, mod, gidx, sidx):
    part = jax.ops.segment_sum(table[gidx] * mod, sidx, num_segments=NN)
    return part, jnp.zeros_like(part)


def _sc_gradrad(gm2, h1, gm1, h, s, t):
    return gm2[t] * h1[s] + gm1[t] * h[s]


def _sc_force(gvec16, s, t):
    part = (jax.ops.segment_sum(gvec16, t, num_segments=NN)
            - jax.ops.segment_sum(gvec16, s, num_segments=NN))
    return part, jnp.zeros_like(part)


def kernel(positions, edge_index, species, W_embed, W_r1, W_r2,
           W_update1, W_update2, w_read, atomic_energies):
    s = edge_index[0].astype(jnp.int32)
    t = edge_index[1].astype(jnp.int32)
    pos16 = jnp.pad(positions.astype(jnp.float32), ((0, 0), (0, 13)))
    sp2d = species.astype(jnp.int32)[:, None]
    we_pad = jnp.pad(W_embed, ((0, 6), (0, 0)))
    ae16 = jnp.pad(atomic_energies, (0, 6))[:, None]
    wr2d = w_read[None, :]

    vec16 = _sc_geom(pos16, s, t)
    radial = _tc_radial(vec16, W_r1, W_r2)
    h = _tc_embed(sp2d, we_pad)
    m1a, m1b = _sc_msg(h, radial, s, t)
    m1, h1 = _tc_node1(m1a, m1b, h, W_update1)
    m2a, m2b = _sc_msg(h1, radial, s, t)
    gm2, en = _tc_node2(m2a, m2b, h1, sp2d, W_update2, wr2d, ae16)
    gsa, gsb = _sc_msg(gm2, radial, t, s)
    gm1 = _tc_gm1(gsa, gsb, m1, W_update1, wr2d)
    g_radial = _sc_gradrad(gm2, h1, gm1, h, s, t)
    gvec16 = _tc_rback(vec16, g_radial, W_r1, W_r2)
    fa, fb = _sc_force(gvec16, s, t)
    forces16 = _tc_forces(fa, fb)
    return en[0, 0], forces16[:, :3]


# speedup vs baseline: 1.7272x; 1.7272x over previous
import functools

import jax
import jax.numpy as jnp
from jax import lax
from jax.experimental import pallas as pl
from jax.experimental.pallas import tpu as pltpu
from jax.experimental.pallas import tpu_sc as plsc

NN = 10000
EE = 320000
DD = 128
DH = 64
NRBF_K = 8
RMAX_V = 10.0
C0 = (2.0 / RMAX_V) ** 0.5

BE = 2000
BN = 2000

NCC = 2
NSS = 16
NWW = NCC * NSS
EPW = EE // NWW
CC = 80
NCH = EPW // CC
RPN = 632
NPAD = RPN * NSS


def _silu(x):
    return x * jax.nn.sigmoid(x)


def _dsilu(x):
    s = jax.nn.sigmoid(x)
    return s * (1.0 + x * (1.0 - s))


def _geom(v):
    d2 = jnp.sum(v * v, axis=1, keepdims=True)
    r = jnp.sqrt(d2 + 1e-12)
    x = r / RMAX_V
    x2 = x * x
    x4 = x2 * x2
    x5 = x4 * x
    x6 = x4 * x2
    x7 = x6 * x
    x8 = x4 * x4
    lt1 = x < 1.0
    env = jnp.where(lt1, 1.0 - 28.0 * x6 + 48.0 * x7 - 21.0 * x8, 0.0)
    denv = jnp.where(lt1, -168.0 * x5 + 336.0 * x6 - 168.0 * x7, 0.0)
    kk = (lax.broadcasted_iota(jnp.int32, (1, NRBF_K), 1) + 1
          ).astype(jnp.float32)
    arg = kk * (jnp.pi * x)
    sinv = jnp.sin(arg)
    rinv = 1.0 / (r + 1e-12)
    rbf = (C0 * env * rinv) * sinv
    return r, env, denv, kk, arg, sinv, rinv, rbf


def _tc_radial_body(vec_ref, w1_ref, w2_ref, lo_ref, hi_ref):
    v = vec_ref[...]
    rbf = _geom(v)[-1]
    z1 = jnp.dot(rbf, w1_ref[...], preferred_element_type=jnp.float32)
    radial = jnp.dot(_silu(z1), w2_ref[...], preferred_element_type=jnp.float32)
    lo_ref[...] = radial[:, :DH]
    hi_ref[...] = radial[:, DH:]


def _tc_rback_body(vec_ref, glo_ref, ghi_ref, w1_ref, w2_ref,
                   outp_ref, outn_ref):
    v = vec_ref[...]
    r, env, denv, kk, arg, sinv, rinv, rbf = _geom(v)
    w1 = w1_ref[...]
    w2 = w2_ref[...]
    z1 = jnp.dot(rbf, w1, preferred_element_type=jnp.float32)
    g_radial = jnp.concatenate([glo_ref[...], ghi_ref[...]], axis=1)
    g_a1 = lax.dot_general(g_radial, w2, (((1,), (1,)), ((), ())),
                           preferred_element_type=jnp.float32)
    g_z1 = g_a1 * _dsilu(z1)
    g_rbf = lax.dot_general(g_z1, w1, (((1,), (1,)), ((), ())),
                            preferred_element_type=jnp.float32)
    cosv = jnp.cos(arg)
    drbf_dr = (C0 * env) * ((kk * (jnp.pi / RMAX_V)) * cosv * rinv
                            - sinv * (rinv * rinv)) \
        + (C0 * sinv * rinv) * (denv / RMAX_V)
    g_r = jnp.sum(g_rbf * drbf_dr, axis=1, keepdims=True)
    gv = (g_r / r) * v[:, :8]
    outp_ref[...] = gv
    outn_ref[...] = -gv


def _tc_embed_body(sp_ref, we_ref, lo_ref, hi_ref):
    sp = sp_ref[...]
    ids = lax.broadcasted_iota(jnp.int32, (1, 16), 1)
    onehot = (sp == ids).astype(jnp.float32)
    h = jnp.dot(onehot, we_ref[...], preferred_element_type=jnp.float32)
    lo_ref[...] = h[:, :DH]
    hi_ref[...] = h[:, DH:]


def _parts(ref_a, ref_b):
    return ref_a[...][0] + ref_b[...][0]


def _tc_node1_body(pla_ref, plb_ref, pha_ref, phb_ref, hlo_ref, hhi_ref,
                   u1_ref, m1_ref, h1lo_ref, h1hi_ref):
    m1 = jnp.concatenate([_parts(pla_ref, plb_ref),
                          _parts(pha_ref, phb_ref)], axis=1)
    h = jnp.concatenate([hlo_ref[...], hhi_ref[...]], axis=1)
    h1 = h + jnp.dot(_silu(m1), u1_ref[...],
                     preferred_element_type=jnp.float32)
    m1_ref[...] = m1
    h1lo_ref[...] = h1[:, :DH]
    h1hi_ref[...] = h1[:, DH:]


def _tc_node2_body(pla_ref, plb_ref, pha_ref, phb_ref, h1lo_ref, h1hi_ref,
                   sp_ref, u2_ref, wr_ref, ae_ref,
                   gm2lo_ref, gm2hi_ref, en_ref):
    m2 = jnp.concatenate([_parts(pla_ref, plb_ref),
                          _parts(pha_ref, phb_ref)], axis=1)
    u2 = u2_ref[...]
    wr = wr_ref[...]
    h1 = jnp.concatenate([h1lo_ref[...], h1hi_ref[...]], axis=1)
    h2 = h1 + jnp.dot(_silu(m2), u2, preferred_element_type=jnp.float32)
    sp = sp_ref[...]
    ids = lax.broadcasted_iota(jnp.int32, (1, 16), 1)
    onehot = (sp == ids).astype(jnp.float32)
    e_block = jnp.sum(h2 * wr) + jnp.sum(
        jnp.dot(onehot, ae_ref[...], preferred_element_type=jnp.float32))
    v2 = lax.dot_general(wr, u2, (((1,), (1,)), ((), ())),
                         preferred_element_type=jnp.float32)
    gm2 = v2 * _dsilu(m2)
    gm2lo_ref[...] = gm2[:, :DH]
    gm2hi_ref[...] = gm2[:, DH:]

    @pl.when(pl.program_id(0) == 0)
    def _():
        en_ref[...] = jnp.zeros_like(en_ref[...])
    en_ref[...] = en_ref[...] + e_block


def _tc_gm1_body(pla_ref, plb_ref, pha_ref, phb_ref, m1_ref, u1_ref, wr_ref,
                 gm1lo_ref, gm1hi_ref):
    gs = jnp.concatenate([_parts(pla_ref, plb_ref),
                          _parts(pha_ref, phb_ref)], axis=1)
    g_h1 = wr_ref[...] + gs
    gm1 = lax.dot_general(
        g_h1, u1_ref[...], (((1,), (1,)), ((), ())),
        preferred_element_type=jnp.float32) * _dsilu(m1_ref[...])
    gm1lo_ref[...] = gm1[:, :DH]
    gm1hi_ref[...] = gm1[:, DH:]


def _tc_forces_body(parts_ref, out_ref):
    out_ref[...] = -jnp.sum(parts_ref[...], axis=0)


def _edge_specs(width):
    return pl.BlockSpec((BE, width), lambda i: (i, 0))


def _node_specs(width):
    return pl.BlockSpec((BN, width), lambda i: (i, 0))


def _full(shape):
    return pl.BlockSpec(shape, lambda i: tuple(0 for _ in shape))


def _part_specs():
    return [pl.BlockSpec((1, BN, DH), lambda i: (0, i, 0)),
            pl.BlockSpec((1, BN, DH), lambda i: (1, i, 0))]


def _tc_radial(vec16, W_r1, W_r2):
    return pl.pallas_call(
        _tc_radial_body,
        grid=(EE // BE,),
        in_specs=[_edge_specs(16), _full((NRBF_K, 64)), _full((64, DD))],
        out_specs=[_edge_specs(DH)] * 2,
        out_shape=[jax.ShapeDtypeStruct((EE, DH), jnp.float32)] * 2,
    )(vec16, W_r1, W_r2)


def _tc_rback(vec16, g_lo, g_hi, W_r1, W_r2):
    return pl.pallas_call(
        _tc_rback_body,
        grid=(EE // BE,),
        in_specs=[_edge_specs(16), _edge_specs(DH), _edge_specs(DH),
                  _full((NRBF_K, 64)), _full((64, DD))],
        out_specs=[_edge_specs(8)] * 2,
        out_shape=[jax.ShapeDtypeStruct((EE, 8), jnp.float32)] * 2,
    )(vec16, g_lo, g_hi, W_r1, W_r2)


def _tc_embed(sp2d, we_pad):
    return pl.pallas_call(
        _tc_embed_body,
        grid=(NN // BN,),
        in_specs=[_node_specs(1), _full((16, DD))],
        out_specs=[_node_specs(DH)] * 2,
        out_shape=[jax.ShapeDtypeStruct((NN, DH), jnp.float32)] * 2,
    )(sp2d, we_pad)


def _tc_node1(plo, phi, h_lo, h_hi, U1):
    return pl.pallas_call(
        _tc_node1_body,
        grid=(NN // BN,),
        in_specs=(_part_specs() + _part_specs()
                  + [_node_specs(DH), _node_specs(DH), _full((DD, DD))]),
        out_specs=[_node_specs(DD), _node_specs(DH), _node_specs(DH)],
        out_shape=[jax.ShapeDtypeStruct((NN, DD), jnp.float32),
                   jax.ShapeDtypeStruct((NN, DH), jnp.float32),
                   jax.ShapeDtypeStruct((NN, DH), jnp.float32)],
    )(plo, plo, phi, phi, h_lo, h_hi, U1)


def _tc_node2(plo, phi, h1_lo, h1_hi, sp2d, U2, wr2d, ae16):
    return pl.pallas_call(
        _tc_node2_body,
        grid=(NN // BN,),
        in_specs=(_part_specs() + _part_specs()
                  + [_node_specs(DH), _node_specs(DH), _node_specs(1),
                     _full((DD, DD)), _full((1, DD)), _full((16, 1))]),
        out_specs=[_node_specs(DH), _node_specs(DH), _full((1, 1))],
        out_shape=[jax.ShapeDtypeStruct((NN, DH), jnp.float32),
                   jax.ShapeDtypeStruct((NN, DH), jnp.float32),
                   jax.ShapeDtypeStruct((1, 1), jnp.float32)],
    )(plo, plo, phi, phi, h1_lo, h1_hi, sp2d, U2, wr2d, ae16)


def _tc_gm1(plo, phi, m1, U1, wr2d):
    return pl.pallas_call(
        _tc_gm1_body,
        grid=(NN // BN,),
        in_specs=(_part_specs() + _part_specs()
                  + [_node_specs(DD), _full((DD, DD)), _full((1, DD))]),
        out_specs=[_node_specs(DH)] * 2,
        out_shape=[jax.ShapeDtypeStruct((NN, DH), jnp.float32)] * 2,
    )(plo, plo, phi, phi, m1, U1, wr2d)


def _tc_forces(parts):
    return pl.pallas_call(
        _tc_forces_body,
        grid=(NN // BN,),
        in_specs=[pl.BlockSpec((NCC, BN, 8), lambda i: (0, i, 0))],
        out_specs=_node_specs(8),
        out_shape=jax.ShapeDtypeStruct((NN, 8), jnp.float32),
    )(parts)


_mesh = plsc.VectorSubcoreMesh(core_axis_name="c", subcore_axis_name="s")
_untiled = pltpu.CompilerParams(use_tc_tiling_on_sc=False)


def _wid():
    return lax.axis_index("c") * NSS + lax.axis_index("s")


@functools.partial(
    pl.kernel,
    out_type=jax.ShapeDtypeStruct((EE, 16), jnp.float32),
    mesh=_mesh,
    scratch_types=[
        pltpu.VMEM((NCH, CC), jnp.int32),
        pltpu.VMEM((NCH, CC), jnp.int32),
        pltpu.VMEM((CC, 16), jnp.float32),
        pltpu.VMEM((CC, 16), jnp.float32),
        pltpu.SemaphoreType.DMA,
    ],
    compiler_params=_untiled,
)
def _sc_geom_kernel(pos_hbm, s_hbm, t_hbm, vec_hbm, s_v, t_v, bufs, buft, sem):
    w = _wid()
    pltpu.sync_copy(s_hbm.at[w], s_v)
    pltpu.sync_copy(t_hbm.at[w], t_v)
    base = w * EPW

    def chunk(j, _):
        pltpu.async_copy(pos_hbm.at[s_v.at[j]], bufs, sem).wait()
        pltpu.async_copy(pos_hbm.at[t_v.at[j]], buft, sem).wait()

        def row(i, _):
            buft[i] = buft[i] - bufs[i]
            return 0
        lax.fori_loop(0, CC, row, 0)
        pltpu.sync_copy(buft, vec_hbm.at[pl.ds(base + j * CC, CC)])
        return 0
    lax.fori_loop(0, NCH, chunk, 0)


@functools.partial(
    pl.kernel,
    out_type=jax.ShapeDtypeStruct((NCC, NPAD, DH), jnp.float32),
    mesh=_mesh,
    scratch_types=[
        pltpu.VMEM((NCH, CC), jnp.int32),
        pltpu.VMEM((NCH, CC), jnp.int32),
        pltpu.VMEM((CC, DH), jnp.float32),
        pltpu.VMEM((CC, DH), jnp.float32),
        pltpu.VMEM_SHARED((NPAD, DH), jnp.float32),
        pltpu.SemaphoreType.DMA,
    ],
    compiler_params=_untiled,
)
def _sc_msg_kernel(table_hbm, mod_hbm, gi_hbm, si_hbm, zero_hbm, out_hbm,
                   gi_v, si_v, rows, mods, acc, sem):
    cid = lax.axis_index("c")
    sub = lax.axis_index("s")
    w = cid * NSS + sub
    r0 = sub * RPN
    pltpu.sync_copy(zero_hbm.at[pl.ds(r0, RPN)], acc.at[pl.ds(r0, RPN)])
    pltpu.sync_copy(gi_hbm.at[w], gi_v)
    pltpu.sync_copy(si_hbm.at[w], si_v)
    plsc.subcore_barrier()
    base = w * EPW

    def chunk(j, _):
        pltpu.async_copy(table_hbm.at[gi_v.at[j]], rows, sem).wait()
        pltpu.sync_copy(mod_hbm.at[pl.ds(base + j * CC, CC)], mods)

        def mrow(i, _):
            def mlane(k, _):
                sl = pl.ds(k * 16, 16)
                rows[i, sl] = rows[i, sl] * mods[i, sl]
                return 0
            lax.fori_loop(0, DH // 16, mlane, 0)
            return 0
        lax.fori_loop(0, CC, mrow, 0)
        pltpu.sync_copy(rows, acc.at[si_v.at[j]], add=True)
        return 0
    lax.fori_loop(0, NCH, chunk, 0)
    plsc.subcore_barrier()
    pltpu.sync_copy(acc.at[pl.ds(r0, RPN)], out_hbm.at[cid, pl.ds(r0, RPN)])


@functools.partial(
    pl.kernel,
    out_type=[jax.ShapeDtypeStruct((EE, DH), jnp.float32)] * 2,
    mesh=_mesh,
    scratch_types=[
        pltpu.VMEM((NCH, CC), jnp.int32),
        pltpu.VMEM((NCH, CC), jnp.int32),
        pltpu.VMEM((CC, DH), jnp.float32),
        pltpu.VMEM((CC, DH), jnp.float32),
        pltpu.VMEM((CC, DH), jnp.float32),
        pltpu.VMEM((CC, DH), jnp.float32),
        pltpu.SemaphoreType.DMA,
    ],
    compiler_params=_untiled,
)
def _sc_gradrad_kernel(gm2lo_hbm, gm2hi_hbm, h1lo_hbm, h1hi_hbm,
                       gm1lo_hbm, gm1hi_hbm, hlo_hbm, hhi_hbm,
                       s_hbm, t_hbm, outlo_hbm, outhi_hbm,
                       s_v, t_v, b1, b2, b3, b4, sem):
    w = _wid()
    pltpu.sync_copy(s_hbm.at[w], s_v)
    pltpu.sync_copy(t_hbm.at[w], t_v)
    base = w * EPW

    def half(gm2_hbm, gm1_hbm, h1_hbm, h_hbm, out_hbm, j):
        pltpu.async_copy(gm2_hbm.at[t_v.at[j]], b1, sem).wait()
        pltpu.async_copy(h1_hbm.at[s_v.at[j]], b2, sem).wait()
        pltpu.async_copy(gm1_hbm.at[t_v.at[j]], b3, sem).wait()
        pltpu.async_copy(h_hbm.at[s_v.at[j]], b4, sem).wait()

        def mrow(i, _):
            def mlane(k, _):
                sl = pl.ds(k * 16, 16)
                b1[i, sl] = b1[i, sl] * b2[i, sl] + b3[i, sl] * b4[i, sl]
                return 0
            lax.fori_loop(0, DH // 16, mlane, 0)
            return 0
        lax.fori_loop(0, CC, mrow, 0)
        pltpu.sync_copy(b1, out_hbm.at[pl.ds(base + j * CC, CC)])

    def chunk(j, _):
        half(gm2lo_hbm, gm1lo_hbm, h1lo_hbm, hlo_hbm, outlo_hbm, j)
        half(gm2hi_hbm, gm1hi_hbm, h1hi_hbm, hhi_hbm, outhi_hbm, j)
        return 0
    lax.fori_loop(0, NCH, chunk, 0)


@functools.partial(
    pl.kernel,
    out_type=jax.ShapeDtypeStruct((NCC, NPAD, 8), jnp.float32),
    mesh=_mesh,
    scratch_types=[
        pltpu.VMEM((NCH, CC), jnp.int32),
        pltpu.VMEM((NCH, CC), jnp.int32),
        pltpu.VMEM((CC, 8), jnp.float32),
        pltpu.VMEM((CC, 8), jnp.float32),
        pltpu.VMEM_SHARED((NPAD, 8), jnp.float32),
    ],
    compiler_params=_untiled,
)
def _sc_force_kernel(gvp_hbm, gvn_hbm, s_hbm, t_hbm, zero_hbm, out_hbm,
                     s_v, t_v, gvp, gvn, acc):
    cid = lax.axis_index("c")
    sub = lax.axis_index("s")
    w = cid * NSS + sub
    r0 = sub * RPN
    pltpu.sync_copy(zero_hbm.at[pl.ds(r0, RPN)], acc.at[pl.ds(r0, RPN)])
    pltpu.sync_copy(s_hbm.at[w], s_v)
    pltpu.sync_copy(t_hbm.at[w], t_v)
    plsc.subcore_barrier()
    base = w * EPW

    def chunk(j, _):
        pltpu.sync_copy(gvp_hbm.at[pl.ds(base + j * CC, CC)], gvp)
        pltpu.sync_copy(gvn_hbm.at[pl.ds(base + j * CC, CC)], gvn)
        pltpu.sync_copy(gvp, acc.at[t_v.at[j]], add=True)
        pltpu.sync_copy(gvn, acc.at[s_v.at[j]], add=True)
        return 0
    lax.fori_loop(0, NCH, chunk, 0)
    plsc.subcore_barrier()
    pltpu.sync_copy(acc.at[pl.ds(r0, RPN)], out_hbm.at[cid, pl.ds(r0, RPN)])


def kernel(positions, edge_index, species, W_embed, W_r1, W_r2,
           W_update1, W_update2, w_read, atomic_energies):
    s3 = edge_index[0].astype(jnp.int32).reshape(NWW, NCH, CC)
    t3 = edge_index[1].astype(jnp.int32).reshape(NWW, NCH, CC)
    pos16 = jnp.pad(positions.astype(jnp.float32), ((0, 0), (0, 13)))
    sp2d = species.astype(jnp.int32)[:, None]
    we_pad = jnp.pad(W_embed, ((0, 6), (0, 0)))
    ae16 = jnp.pad(atomic_energies, (0, 6))[:, None]
    wr2d = w_read[None, :]
    zeros_nd = jnp.zeros((NPAD, DH), jnp.float32)
    zeros_n8 = jnp.zeros((NPAD, 8), jnp.float32)

    vec16 = _sc_geom_kernel(pos16, s3, t3)
    rad_lo, rad_hi = _tc_radial(vec16, W_r1, W_r2)
    h_lo, h_hi = _tc_embed(sp2d, we_pad)
    m1plo = _sc_msg_kernel(h_lo, rad_lo, s3, t3, zeros_nd)
    m1phi = _sc_msg_kernel(h_hi, rad_hi, s3, t3, zeros_nd)
    m1, h1_lo, h1_hi = _tc_node1(m1plo, m1phi, h_lo, h_hi, W_update1)
    m2plo = _sc_msg_kernel(h1_lo, rad_lo, s3, t3, zeros_nd)
    m2phi = _sc_msg_kernel(h1_hi, rad_hi, s3, t3, zeros_nd)
    gm2_lo, gm2_hi, en = _tc_node2(m2plo, m2phi, h1_lo, h1_hi, sp2d,
                                   W_update2, wr2d, ae16)
    gsplo = _sc_msg_kernel(gm2_lo, rad_lo, t3, s3, zeros_nd)
    gsphi = _sc_msg_kernel(gm2_hi, rad_hi, t3, s3, zeros_nd)
    gm1_lo, gm1_hi = _tc_gm1(gsplo, gsphi, m1, W_update1, wr2d)
    g_lo, g_hi = _sc_gradrad_kernel(gm2_lo, gm2_hi, h1_lo, h1_hi,
                                    gm1_lo, gm1_hi, h_lo, h_hi, s3, t3)
    gvp, gvn = _tc_rback(vec16, g_lo, g_hi, W_r1, W_r2)
    fparts = _sc_force_kernel(gvp, gvn, s3, t3, zeros_n8)
    forces8 = _tc_forces(fparts)
    return en[0, 0], forces8[:, :3]


# speedup vs baseline: 2.6159x; 1.5146x over previous
import functools

import jax
import jax.numpy as jnp
from jax import lax
from jax.experimental import pallas as pl
from jax.experimental.pallas import tpu as pltpu
from jax.experimental.pallas import tpu_sc as plsc

NN = 10000
EE = 320000
DD = 128
DH = 64
NRBF_K = 8
RMAX_V = 10.0
C0 = (2.0 / RMAX_V) ** 0.5

BE = 2000
BN = 2000

NCC = 2
NSS = 16
NWW = NCC * NSS
EPW = EE // NWW
CC = 80
NCH = EPW // CC
RPN = 632
NPAD = RPN * NSS


def _silu(x):
    return x * jax.nn.sigmoid(x)


def _dsilu(x):
    s = jax.nn.sigmoid(x)
    return s * (1.0 + x * (1.0 - s))


def _geom(v):
    d2 = jnp.sum(v * v, axis=1, keepdims=True)
    r = jnp.sqrt(d2 + 1e-12)
    x = r / RMAX_V
    x2 = x * x
    x4 = x2 * x2
    x5 = x4 * x
    x6 = x4 * x2
    x7 = x6 * x
    x8 = x4 * x4
    lt1 = x < 1.0
    env = jnp.where(lt1, 1.0 - 28.0 * x6 + 48.0 * x7 - 21.0 * x8, 0.0)
    denv = jnp.where(lt1, -168.0 * x5 + 336.0 * x6 - 168.0 * x7, 0.0)
    kk = (lax.broadcasted_iota(jnp.int32, (1, NRBF_K), 1) + 1
          ).astype(jnp.float32)
    arg = kk * (jnp.pi * x)
    sinv = jnp.sin(arg)
    rinv = 1.0 / (r + 1e-12)
    rbf = (C0 * env * rinv) * sinv
    return r, env, denv, kk, arg, sinv, rinv, rbf


def _tc_radial_body(vec_ref, w1_ref, w2_ref, lo_ref, hi_ref):
    v = vec_ref[...]
    rbf = _geom(v)[-1]
    z1 = jnp.dot(rbf, w1_ref[...], preferred_element_type=jnp.float32)
    radial = jnp.dot(_silu(z1), w2_ref[...], preferred_element_type=jnp.float32)
    lo_ref[...] = radial[:, :DH]
    hi_ref[...] = radial[:, DH:]


def _tc_rback_body(vec_ref, glo_ref, ghi_ref, w1_ref, w2_ref,
                   outp_ref, outn_ref):
    v = vec_ref[...]
    r, env, denv, kk, arg, sinv, rinv, rbf = _geom(v)
    w1 = w1_ref[...]
    w2 = w2_ref[...]
    z1 = jnp.dot(rbf, w1, preferred_element_type=jnp.float32)
    g_radial = jnp.concatenate([glo_ref[...], ghi_ref[...]], axis=1)
    g_a1 = lax.dot_general(g_radial, w2, (((1,), (1,)), ((), ())),
                           preferred_element_type=jnp.float32)
    g_z1 = g_a1 * _dsilu(z1)
    g_rbf = lax.dot_general(g_z1, w1, (((1,), (1,)), ((), ())),
                            preferred_element_type=jnp.float32)
    cosv = jnp.cos(arg)
    drbf_dr = (C0 * env) * ((kk * (jnp.pi / RMAX_V)) * cosv * rinv
                            - sinv * (rinv * rinv)) \
        + (C0 * sinv * rinv) * (denv / RMAX_V)
    g_r = jnp.sum(g_rbf * drbf_dr, axis=1, keepdims=True)
    gv = (g_r / r) * v[:, :8]
    outp_ref[...] = gv
    outn_ref[...] = -gv


def _tc_embed_body(sp_ref, we_ref, lo_ref, hi_ref):
    sp = sp_ref[...]
    ids = lax.broadcasted_iota(jnp.int32, (1, 16), 1)
    onehot = (sp == ids).astype(jnp.float32)
    h = jnp.dot(onehot, we_ref[...], preferred_element_type=jnp.float32)
    lo_ref[...] = h[:, :DH]
    hi_ref[...] = h[:, DH:]


def _parts(ref_a, ref_b):
    return ref_a[...][0] + ref_b[...][0]


def _tc_node1_body(pla_ref, plb_ref, pha_ref, phb_ref, hlo_ref, hhi_ref,
                   u1_ref, m1_ref, h1lo_ref, h1hi_ref):
    m1 = jnp.concatenate([_parts(pla_ref, plb_ref),
                          _parts(pha_ref, phb_ref)], axis=1)
    h = jnp.concatenate([hlo_ref[...], hhi_ref[...]], axis=1)
    h1 = h + jnp.dot(_silu(m1), u1_ref[...],
                     preferred_element_type=jnp.float32)
    m1_ref[...] = m1
    h1lo_ref[...] = h1[:, :DH]
    h1hi_ref[...] = h1[:, DH:]


def _tc_node2_body(pla_ref, plb_ref, pha_ref, phb_ref, h1lo_ref, h1hi_ref,
                   sp_ref, u2_ref, wr_ref, ae_ref,
                   gm2lo_ref, gm2hi_ref, en_ref):
    m2 = jnp.concatenate([_parts(pla_ref, plb_ref),
                          _parts(pha_ref, phb_ref)], axis=1)
    u2 = u2_ref[...]
    wr = wr_ref[...]
    h1 = jnp.concatenate([h1lo_ref[...], h1hi_ref[...]], axis=1)
    h2 = h1 + jnp.dot(_silu(m2), u2, preferred_element_type=jnp.float32)
    sp = sp_ref[...]
    ids = lax.broadcasted_iota(jnp.int32, (1, 16), 1)
    onehot = (sp == ids).astype(jnp.float32)
    e_block = jnp.sum(h2 * wr) + jnp.sum(
        jnp.dot(onehot, ae_ref[...], preferred_element_type=jnp.float32))
    v2 = lax.dot_general(wr, u2, (((1,), (1,)), ((), ())),
                         preferred_element_type=jnp.float32)
    gm2 = v2 * _dsilu(m2)
    gm2lo_ref[...] = gm2[:, :DH]
    gm2hi_ref[...] = gm2[:, DH:]

    @pl.when(pl.program_id(0) == 0)
    def _():
        en_ref[...] = jnp.zeros_like(en_ref[...])
    en_ref[...] = en_ref[...] + e_block


def _tc_gm1_body(pla_ref, plb_ref, pha_ref, phb_ref, m1_ref, u1_ref, wr_ref,
                 gm1lo_ref, gm1hi_ref):
    gs = jnp.concatenate([_parts(pla_ref, plb_ref),
                          _parts(pha_ref, phb_ref)], axis=1)
    g_h1 = wr_ref[...] + gs
    gm1 = lax.dot_general(
        g_h1, u1_ref[...], (((1,), (1,)), ((), ())),
        preferred_element_type=jnp.float32) * _dsilu(m1_ref[...])
    gm1lo_ref[...] = gm1[:, :DH]
    gm1hi_ref[...] = gm1[:, DH:]


def _tc_forces_body(parts_ref, out_ref):
    out_ref[...] = -jnp.sum(parts_ref[...], axis=0)


def _edge_specs(width):
    return pl.BlockSpec((BE, width), lambda i: (i, 0))


def _node_specs(width):
    return pl.BlockSpec((BN, width), lambda i: (i, 0))


def _full(shape):
    return pl.BlockSpec(shape, lambda i: tuple(0 for _ in shape))


def _part_specs():
    return [pl.BlockSpec((1, BN, DH), lambda i: (0, i, 0)),
            pl.BlockSpec((1, BN, DH), lambda i: (1, i, 0))]


def _tc_radial(vec16, W_r1, W_r2):
    return pl.pallas_call(
        _tc_radial_body,
        grid=(EE // BE,),
        in_specs=[_edge_specs(16), _full((NRBF_K, 64)), _full((64, DD))],
        out_specs=[_edge_specs(DH)] * 2,
        out_shape=[jax.ShapeDtypeStruct((EE, DH), jnp.float32)] * 2,
    )(vec16, W_r1, W_r2)


def _tc_rback(vec16, g_lo, g_hi, W_r1, W_r2):
    return pl.pallas_call(
        _tc_rback_body,
        grid=(EE // BE,),
        in_specs=[_edge_specs(16), _edge_specs(DH), _edge_specs(DH),
                  _full((NRBF_K, 64)), _full((64, DD))],
        out_specs=[_edge_specs(8)] * 2,
        out_shape=[jax.ShapeDtypeStruct((EE, 8), jnp.float32)] * 2,
    )(vec16, g_lo, g_hi, W_r1, W_r2)


def _tc_embed(sp2d, we_pad):
    return pl.pallas_call(
        _tc_embed_body,
        grid=(NN // BN,),
        in_specs=[_node_specs(1), _full((16, DD))],
        out_specs=[_node_specs(DH)] * 2,
        out_shape=[jax.ShapeDtypeStruct((NN, DH), jnp.float32)] * 2,
    )(sp2d, we_pad)


def _tc_node1(plo, phi, h_lo, h_hi, U1):
    return pl.pallas_call(
        _tc_node1_body,
        grid=(NN // BN,),
        in_specs=(_part_specs() + _part_specs()
                  + [_node_specs(DH), _node_specs(DH), _full((DD, DD))]),
        out_specs=[_node_specs(DD), _node_specs(DH), _node_specs(DH)],
        out_shape=[jax.ShapeDtypeStruct((NN, DD), jnp.float32),
                   jax.ShapeDtypeStruct((NN, DH), jnp.float32),
                   jax.ShapeDtypeStruct((NN, DH), jnp.float32)],
    )(plo, plo, phi, phi, h_lo, h_hi, U1)


def _tc_node2(plo, phi, h1_lo, h1_hi, sp2d, U2, wr2d, ae16):
    return pl.pallas_call(
        _tc_node2_body,
        grid=(NN // BN,),
        in_specs=(_part_specs() + _part_specs()
                  + [_node_specs(DH), _node_specs(DH), _node_specs(1),
                     _full((DD, DD)), _full((1, DD)), _full((16, 1))]),
        out_specs=[_node_specs(DH), _node_specs(DH), _full((1, 1))],
        out_shape=[jax.ShapeDtypeStruct((NN, DH), jnp.float32),
                   jax.ShapeDtypeStruct((NN, DH), jnp.float32),
                   jax.ShapeDtypeStruct((1, 1), jnp.float32)],
    )(plo, plo, phi, phi, h1_lo, h1_hi, sp2d, U2, wr2d, ae16)


def _tc_gm1(plo, phi, m1, U1, wr2d):
    return pl.pallas_call(
        _tc_gm1_body,
        grid=(NN // BN,),
        in_specs=(_part_specs() + _part_specs()
                  + [_node_specs(DD), _full((DD, DD)), _full((1, DD))]),
        out_specs=[_node_specs(DH)] * 2,
        out_shape=[jax.ShapeDtypeStruct((NN, DH), jnp.float32)] * 2,
    )(plo, plo, phi, phi, m1, U1, wr2d)


def _tc_forces(parts):
    return pl.pallas_call(
        _tc_forces_body,
        grid=(NN // BN,),
        in_specs=[pl.BlockSpec((NCC, BN, 8), lambda i: (0, i, 0))],
        out_specs=_node_specs(8),
        out_shape=jax.ShapeDtypeStruct((NN, 8), jnp.float32),
    )(parts)


_mesh = plsc.VectorSubcoreMesh(core_axis_name="c", subcore_axis_name="s")
_untiled = pltpu.CompilerParams(use_tc_tiling_on_sc=False)


def _wid():
    return lax.axis_index("c") * NSS + lax.axis_index("s")


@functools.partial(
    pl.kernel,
    out_type=jax.ShapeDtypeStruct((EE, 16), jnp.float32),
    mesh=_mesh,
    scratch_types=[
        pltpu.VMEM((NCH, CC), jnp.int32),
        pltpu.VMEM((NCH, CC), jnp.int32),
        pltpu.VMEM((CC, 16), jnp.float32),
        pltpu.VMEM((CC, 16), jnp.float32),
        pltpu.VMEM((CC, 16), jnp.float32),
        pltpu.VMEM((CC, 16), jnp.float32),
        pltpu.VMEM((CC, 16), jnp.float32),
        pltpu.VMEM((CC, 16), jnp.float32),
        pltpu.SemaphoreType.DMA,
        pltpu.SemaphoreType.DMA,
        pltpu.SemaphoreType.DMA,
        pltpu.SemaphoreType.DMA,
    ],
    compiler_params=_untiled,
)
def _sc_geom_kernel(pos_hbm, s_hbm, t_hbm, vec_hbm, s_v, t_v,
                    bufs0, bufs1, buft0, buft1, vb0, vb1,
                    gsem0, gsem1, wsem0, wsem1):
    w = _wid()
    pltpu.sync_copy(s_hbm.at[w], s_v)
    pltpu.sync_copy(t_hbm.at[w], t_v)
    base = w * EPW

    def fetch(j, bufs, buft, gsem):
        pltpu.async_copy(pos_hbm.at[s_v.at[j]], bufs, gsem)
        pltpu.async_copy(pos_hbm.at[t_v.at[j]], buft, gsem)

    fetch(0, bufs0, buft0, gsem0)
    fetch(1, bufs1, buft1, gsem1)

    def process(j, bufs, buft, vb, gsem, wsem):
        pltpu.make_async_copy(pos_hbm.at[s_v.at[j]], bufs, gsem).wait()
        pltpu.make_async_copy(pos_hbm.at[t_v.at[j]], buft, gsem).wait()

        @pl.when(j >= 2)
        def _():
            pltpu.make_async_copy(
                vb, vec_hbm.at[pl.ds(base, CC)], wsem).wait()

        def row(i, _):
            vb[i] = buft[i] - bufs[i]
            return 0
        lax.fori_loop(0, CC, row, 0)

        @pl.when(j + 2 < NCH)
        def _():
            fetch(j + 2, bufs, buft, gsem)
        pltpu.async_copy(vb, vec_hbm.at[pl.ds(base + j * CC, CC)], wsem)

    def body(j, _):
        @pl.when(j % 2 == 0)
        def _():
            process(j, bufs0, buft0, vb0, gsem0, wsem0)

        @pl.when(j % 2 == 1)
        def _():
            process(j, bufs1, buft1, vb1, gsem1, wsem1)
        return 0
    lax.fori_loop(0, NCH, body, 0)
    pltpu.make_async_copy(vb0, vec_hbm.at[pl.ds(base, CC)], wsem0).wait()
    pltpu.make_async_copy(vb1, vec_hbm.at[pl.ds(base, CC)], wsem1).wait()


@functools.partial(
    pl.kernel,
    out_type=jax.ShapeDtypeStruct((NCC, NPAD, DH), jnp.float32),
    mesh=_mesh,
    scratch_types=[
        pltpu.VMEM((NCH, CC), jnp.int32),
        pltpu.VMEM((NCH, CC), jnp.int32),
        pltpu.VMEM((CC, DH), jnp.float32),
        pltpu.VMEM((CC, DH), jnp.float32),
        pltpu.VMEM((CC, DH), jnp.float32),
        pltpu.VMEM((CC, DH), jnp.float32),
        pltpu.VMEM((CC, DH), jnp.float32),
        pltpu.VMEM((CC, DH), jnp.float32),
        pltpu.VMEM_SHARED((NPAD, DH), jnp.float32),
        pltpu.SemaphoreType.DMA,
        pltpu.SemaphoreType.DMA,
        pltpu.SemaphoreType.DMA,
        pltpu.SemaphoreType.DMA,
        pltpu.SemaphoreType.DMA,
        pltpu.SemaphoreType.DMA,
    ],
    compiler_params=_untiled,
)
def _sc_msg_kernel(table_hbm, mod_hbm, gi_hbm, si_hbm, zero_hbm, out_hbm,
                   gi_v, si_v, rows0, rows1, mods0, mods1, prod0, prod1,
                   acc, gsem0, gsem1, msem0, msem1, ssem0, ssem1):
    cid = lax.axis_index("c")
    sub = lax.axis_index("s")
    w = cid * NSS + sub
    r0 = sub * RPN
    pltpu.sync_copy(zero_hbm.at[pl.ds(r0, RPN)], acc.at[pl.ds(r0, RPN)])
    pltpu.sync_copy(gi_hbm.at[w], gi_v)
    pltpu.sync_copy(si_hbm.at[w], si_v)
    plsc.subcore_barrier()
    base = w * EPW

    def fetch(j, rows, mods, gsem, msem):
        pltpu.async_copy(table_hbm.at[gi_v.at[j]], rows, gsem)
        pltpu.async_copy(mod_hbm.at[pl.ds(base + j * CC, CC)], mods, msem)

    fetch(0, rows0, mods0, gsem0, msem0)
    fetch(1, rows1, mods1, gsem1, msem1)

    def process(j, rows, mods, prod, gsem, msem, ssem):
        pltpu.make_async_copy(table_hbm.at[gi_v.at[j]], rows, gsem).wait()
        pltpu.make_async_copy(
            mod_hbm.at[pl.ds(base + j * CC, CC)], mods, msem).wait()

        @pl.when(j >= 2)
        def _():
            pltpu.make_async_copy(prod, acc.at[si_v.at[j]], ssem).wait()

        def mrow(i, _):
            def mlane(k, _):
                sl = pl.ds(k * 16, 16)
                prod[i, sl] = rows[i, sl] * mods[i, sl]
                return 0
            lax.fori_loop(0, DH // 16, mlane, 0)
            return 0
        lax.fori_loop(0, CC, mrow, 0)

        @pl.when(j + 2 < NCH)
        def _():
            fetch(j + 2, rows, mods, gsem, msem)
        pltpu.make_async_copy(prod, acc.at[si_v.at[j]], ssem).start(add=True)

    def body(j, _):
        @pl.when(j % 2 == 0)
        def _():
            process(j, rows0, mods0, prod0, gsem0, msem0, ssem0)

        @pl.when(j % 2 == 1)
        def _():
            process(j, rows1, mods1, prod1, gsem1, msem1, ssem1)
        return 0
    lax.fori_loop(0, NCH, body, 0)
    pltpu.make_async_copy(prod0, acc.at[si_v.at[0]], ssem0).wait()
    pltpu.make_async_copy(prod1, acc.at[si_v.at[0]], ssem1).wait()
    plsc.subcore_barrier()
    pltpu.sync_copy(acc.at[pl.ds(r0, RPN)], out_hbm.at[cid, pl.ds(r0, RPN)])


@functools.partial(
    pl.kernel,
    out_type=[jax.ShapeDtypeStruct((EE, DH), jnp.float32)] * 2,
    mesh=_mesh,
    scratch_types=[
        pltpu.VMEM((NCH, CC), jnp.int32),
        pltpu.VMEM((NCH, CC), jnp.int32),
        pltpu.VMEM((CC, DH), jnp.float32),
        pltpu.VMEM((CC, DH), jnp.float32),
        pltpu.VMEM((CC, DH), jnp.float32),
        pltpu.VMEM((CC, DH), jnp.float32),
        pltpu.VMEM((CC, DH), jnp.float32),
        pltpu.VMEM((CC, DH), jnp.float32),
        pltpu.VMEM((CC, DH), jnp.float32),
        pltpu.VMEM((CC, DH), jnp.float32),
        pltpu.VMEM((CC, DH), jnp.float32),
        pltpu.VMEM((CC, DH), jnp.float32),
        pltpu.SemaphoreType.DMA,
        pltpu.SemaphoreType.DMA,
    ],
    compiler_params=_untiled,
)
def _sc_gradrad_kernel(gm2lo_hbm, gm2hi_hbm, h1lo_hbm, h1hi_hbm,
                       gm1lo_hbm, gm1hi_hbm, hlo_hbm, hhi_hbm,
                       s_hbm, t_hbm, outlo_hbm, outhi_hbm,
                       s_v, t_v, a1, a2, a3, a4, b1, b2, b3, b4,
                       olo, ohi, gsem, wsem):
    w = _wid()
    pltpu.sync_copy(s_hbm.at[w], s_v)
    pltpu.sync_copy(t_hbm.at[w], t_v)
    base = w * EPW

    def chunk(j, _):
        pltpu.async_copy(gm2lo_hbm.at[t_v.at[j]], a1, gsem)
        pltpu.async_copy(h1lo_hbm.at[s_v.at[j]], a2, gsem)
        pltpu.async_copy(gm1lo_hbm.at[t_v.at[j]], a3, gsem)
        pltpu.async_copy(hlo_hbm.at[s_v.at[j]], a4, gsem)
        pltpu.async_copy(gm2hi_hbm.at[t_v.at[j]], b1, gsem)
        pltpu.async_copy(h1hi_hbm.at[s_v.at[j]], b2, gsem)
        pltpu.async_copy(gm1hi_hbm.at[t_v.at[j]], b3, gsem)
        pltpu.async_copy(hhi_hbm.at[s_v.at[j]], b4, gsem)
        for buf in (a1, a2, a3, a4, b1, b2, b3, b4):
            pltpu.make_async_copy(hlo_hbm.at[s_v.at[j]], buf, gsem).wait()

        @pl.when(j >= 1)
        def _():
            pltpu.make_async_copy(
                olo, outlo_hbm.at[pl.ds(base, CC)], wsem).wait()
            pltpu.make_async_copy(
                ohi, outhi_hbm.at[pl.ds(base, CC)], wsem).wait()

        def mrow(i, _):
            def mlane(k, _):
                sl = pl.ds(k * 16, 16)
                olo[i, sl] = a1[i, sl] * a2[i, sl] + a3[i, sl] * a4[i, sl]
                ohi[i, sl] = b1[i, sl] * b2[i, sl] + b3[i, sl] * b4[i, sl]
                return 0
            lax.fori_loop(0, DH // 16, mlane, 0)
            return 0
        lax.fori_loop(0, CC, mrow, 0)
        pltpu.async_copy(olo, outlo_hbm.at[pl.ds(base + j * CC, CC)], wsem)
        pltpu.async_copy(ohi, outhi_hbm.at[pl.ds(base + j * CC, CC)], wsem)
        return 0
    lax.fori_loop(0, NCH, chunk, 0)
    pltpu.make_async_copy(olo, outlo_hbm.at[pl.ds(base, CC)], wsem).wait()
    pltpu.make_async_copy(ohi, outhi_hbm.at[pl.ds(base, CC)], wsem).wait()


@functools.partial(
    pl.kernel,
    out_type=jax.ShapeDtypeStruct((NCC, NPAD, 8), jnp.float32),
    mesh=_mesh,
    scratch_types=[
        pltpu.VMEM((NCH, CC), jnp.int32),
        pltpu.VMEM((NCH, CC), jnp.int32),
        pltpu.VMEM((CC, 8), jnp.float32),
        pltpu.VMEM((CC, 8), jnp.float32),
        pltpu.VMEM((CC, 8), jnp.float32),
        pltpu.VMEM((CC, 8), jnp.float32),
        pltpu.VMEM_SHARED((NPAD, 8), jnp.float32),
        pltpu.SemaphoreType.DMA,
        pltpu.SemaphoreType.DMA,
        pltpu.SemaphoreType.DMA,
        pltpu.SemaphoreType.DMA,
    ],
    compiler_params=_untiled,
)
def _sc_force_kernel(gvp_hbm, gvn_hbm, s_hbm, t_hbm, zero_hbm, out_hbm,
                     s_v, t_v, gvp0, gvp1, gvn0, gvn1, acc,
                     rsem0, rsem1, ssem0, ssem1):
    cid = lax.axis_index("c")
    sub = lax.axis_index("s")
    w = cid * NSS + sub
    r0 = sub * RPN
    pltpu.sync_copy(zero_hbm.at[pl.ds(r0, RPN)], acc.at[pl.ds(r0, RPN)])
    pltpu.sync_copy(s_hbm.at[w], s_v)
    pltpu.sync_copy(t_hbm.at[w], t_v)
    plsc.subcore_barrier()
    base = w * EPW

    def fetch(j, gvp, gvn, rsem):
        pltpu.async_copy(gvp_hbm.at[pl.ds(base + j * CC, CC)], gvp, rsem)
        pltpu.async_copy(gvn_hbm.at[pl.ds(base + j * CC, CC)], gvn, rsem)

    fetch(0, gvp0, gvn0, rsem0)
    fetch(1, gvp1, gvn1, rsem1)

    def process(j, gvp, gvn, rsem, ssem):
        pltpu.make_async_copy(
            gvp_hbm.at[pl.ds(base, CC)], gvp, rsem).wait()
        pltpu.make_async_copy(
            gvn_hbm.at[pl.ds(base, CC)], gvn, rsem).wait()
        pltpu.make_async_copy(gvp, acc.at[t_v.at[j]], ssem).start(add=True)
        pltpu.make_async_copy(gvn, acc.at[s_v.at[j]], ssem).start(add=True)

        @pl.when(j + 2 < NCH)
        def _():
            pltpu.make_async_copy(gvp, acc.at[t_v.at[j]], ssem).wait()
            pltpu.make_async_copy(gvn, acc.at[s_v.at[j]], ssem).wait()
            fetch(j + 2, gvp, gvn, rsem)

    def body(j, _):
        @pl.when(j % 2 == 0)
        def _():
            process(j, gvp0, gvn0, rsem0, ssem0)

        @pl.when(j % 2 == 1)
        def _():
            process(j, gvp1, gvn1, rsem1, ssem1)
        return 0
    lax.fori_loop(0, NCH, body, 0)
    pltpu.make_async_copy(gvp0, acc.at[t_v.at[0]], ssem0).wait()
    pltpu.make_async_copy(gvn0, acc.at[s_v.at[0]], ssem0).wait()
    pltpu.make_async_copy(gvp1, acc.at[t_v.at[0]], ssem1).wait()
    pltpu.make_async_copy(gvn1, acc.at[s_v.at[0]], ssem1).wait()
    plsc.subcore_barrier()
    pltpu.sync_copy(acc.at[pl.ds(r0, RPN)], out_hbm.at[cid, pl.ds(r0, RPN)])


def kernel(positions, edge_index, species, W_embed, W_r1, W_r2,
           W_update1, W_update2, w_read, atomic_energies):
    s3 = edge_index[0].astype(jnp.int32).reshape(NWW, NCH, CC)
    t3 = edge_index[1].astype(jnp.int32).reshape(NWW, NCH, CC)
    pos16 = jnp.pad(positions.astype(jnp.float32), ((0, 0), (0, 13)))
    sp2d = species.astype(jnp.int32)[:, None]
    we_pad = jnp.pad(W_embed, ((0, 6), (0, 0)))
    ae16 = jnp.pad(atomic_energies, (0, 6))[:, None]
    wr2d = w_read[None, :]
    zeros_nd = jnp.zeros((NPAD, DH), jnp.float32)
    zeros_n8 = jnp.zeros((NPAD, 8), jnp.float32)

    vec16 = _sc_geom_kernel(pos16, s3, t3)
    rad_lo, rad_hi = _tc_radial(vec16, W_r1, W_r2)
    h_lo, h_hi = _tc_embed(sp2d, we_pad)
    m1plo = _sc_msg_kernel(h_lo, rad_lo, s3, t3, zeros_nd)
    m1phi = _sc_msg_kernel(h_hi, rad_hi, s3, t3, zeros_nd)
    m1, h1_lo, h1_hi = _tc_node1(m1plo, m1phi, h_lo, h_hi, W_update1)
    m2plo = _sc_msg_kernel(h1_lo, rad_lo, s3, t3, zeros_nd)
    m2phi = _sc_msg_kernel(h1_hi, rad_hi, s3, t3, zeros_nd)
    gm2_lo, gm2_hi, en = _tc_node2(m2plo, m2phi, h1_lo, h1_hi, sp2d,
                                   W_update2, wr2d, ae16)
    gsplo = _sc_msg_kernel(gm2_lo, rad_lo, t3, s3, zeros_nd)
    gsphi = _sc_msg_kernel(gm2_hi, rad_hi, t3, s3, zeros_nd)
    gm1_lo, gm1_hi = _tc_gm1(gsplo, gsphi, m1, W_update1, wr2d)
    g_lo, g_hi = _sc_gradrad_kernel(gm2_lo, gm2_hi, h1_lo, h1_hi,
                                    gm1_lo, gm1_hi, h_lo, h_hi, s3, t3)
    gvp, gvn = _tc_rback(vec16, g_lo, g_hi, W_r1, W_r2)
    fparts = _sc_force_kernel(gvp, gvn, s3, t3, zeros_n8)
    forces8 = _tc_forces(fparts)
    return en[0, 0], forces8[:, :3]
